# Initial kernel scaffold; baseline (speedup 1.0000x reference)
import functools

import jax
import jax.numpy as jnp
from jax import lax
from jax.experimental import pallas as pl
from jax.experimental.pallas import tpu as pltpu
from jax.experimental.pallas import tpu_sc as plsc

NC = 2
NS = 16
NW = NC * NS
CH = 128


def _hist_body(ncb, ncr, seg, edges, ones_h, zeros_h, out, sidx, didx, ones_v,
               acc_s, acc_d):
    cid = lax.axis_index("c")
    sid = lax.axis_index("s")
    wid = sid * NC + cid

    pltpu.sync_copy(ones_h, ones_v)
    pltpu.sync_copy(zeros_h, acc_s.at[pl.ds(sid * seg, seg)])
    pltpu.sync_copy(zeros_h, acc_d.at[pl.ds(sid * seg, seg)])
    plsc.subcore_barrier()

    nchunks = jnp.where(wid < ncr, ncb + 1, ncb)

    def body(i, carry):
        base = (wid + NW * i) * CH
        pltpu.sync_copy(edges.at[0, pl.ds(base, CH)], sidx.at[0])
        pltpu.sync_copy(edges.at[1, pl.ds(base, CH)], didx.at[0])
        pltpu.sync_copy(ones_v, acc_s.at[sidx.at[0]], add=True)
        pltpu.sync_copy(ones_v, acc_d.at[didx.at[0]], add=True)
        return carry

    lax.fori_loop(0, nchunks, body, 0)
    plsc.subcore_barrier()
    pltpu.sync_copy(acc_s.at[pl.ds(sid * seg, seg)],
                    out.at[cid, 0, pl.ds(sid * seg, seg)])
    pltpu.sync_copy(acc_d.at[pl.ds(sid * seg, seg)],
                    out.at[cid, 1, pl.ds(sid * seg, seg)])


def _agg_body(ncb, ncr, rpt, h, edges, zeros_a, out, sidx, didx, rows, acc,
              sem):
    cid = lax.axis_index("c")
    sid = lax.axis_index("s")
    wid = sid * NC + cid

    pltpu.sync_copy(zeros_a, acc.at[pl.ds(sid * rpt, rpt)])
    plsc.subcore_barrier()

    nchunks = jnp.where(wid < ncr, ncb + 1, ncb)

    def body(i, carry):
        base = (wid + NW * i) * CH
        pltpu.sync_copy(edges.at[0, pl.ds(base, CH)], sidx.at[0])
        pltpu.sync_copy(edges.at[1, pl.ds(base, CH)], didx.at[0])
        pltpu.async_copy(h.at[sidx.at[0]], rows.at[0], sem).wait()
        pltpu.sync_copy(rows.at[0], acc.at[didx.at[0]], add=True)
        return carry

    lax.fori_loop(0, nchunks, body, 0)
    plsc.subcore_barrier()
    pltpu.sync_copy(acc.at[pl.ds(sid * rpt, rpt)],
                    out.at[cid, pl.ds(sid * rpt, rpt)])


def _mm_body(deg_ref, x_ref, w_ref, o_ref):
    norm = lax.rsqrt(jnp.maximum(deg_ref[...], 1.0))
    o_ref[...] = jnp.dot(x_ref[...] * norm, w_ref[...],
                         preferred_element_type=jnp.float32)


def _fin_body(p_ref, deg_ref, b_ref, x_ref, o_ref):
    agg = p_ref[0] + p_ref[1]
    norm = lax.rsqrt(jnp.maximum(deg_ref[...], 1.0))
    o = jnp.maximum(agg * norm + b_ref[...], 0.0)
    o_ref[...] = o + x_ref[...]


def kernel(feats, edge_index, W, b):
    n, d = feats.shape
    e = edge_index.shape[1]
    assert e % CH == 0 and n % NS == 0
    ec = e // CH
    ncb, ncr = ec // NW, ec % NW
    seg = ((n + NS * 8 - 1) // (NS * 8)) * 8
    npad = NS * seg
    rpt = n // NS

    mesh = plsc.VectorSubcoreMesh(core_axis_name="c", subcore_axis_name="s")

    hist = pl.kernel(
        functools.partial(_hist_body, ncb, ncr, seg),
        mesh=mesh,
        out_type=jax.ShapeDtypeStruct((NC, 2, npad), jnp.float32),
        scratch_types=[
            pltpu.VMEM((1, CH), jnp.int32),
            pltpu.VMEM((1, CH), jnp.int32),
            pltpu.VMEM((CH,), jnp.float32),
            pltpu.VMEM_SHARED((npad,), jnp.float32),
            pltpu.VMEM_SHARED((npad,), jnp.float32),
        ],
    )(edge_index, jnp.ones((CH,), jnp.float32), jnp.zeros((seg,), jnp.float32))

    deg_src = (hist[0, 0, :n] + hist[1, 0, :n]).reshape(n, 1)
    deg_dst = (hist[0, 1, :n] + hist[1, 1, :n]).reshape(n, 1)

    bm = 2000
    assert n % bm == 0
    h = pl.pallas_call(
        _mm_body,
        grid=(n // bm,),
        in_specs=[
            pl.BlockSpec((bm, 1), lambda i: (i, 0)),
            pl.BlockSpec((bm, d), lambda i: (i, 0)),
            pl.BlockSpec((d, d), lambda i: (0, 0)),
        ],
        out_specs=pl.BlockSpec((bm, d), lambda i: (i, 0)),
        out_shape=jax.ShapeDtypeStruct((n, d), jnp.float32),
    )(deg_src, feats, W)

    parts = pl.kernel(
        functools.partial(_agg_body, ncb, ncr, rpt),
        mesh=mesh,
        out_type=jax.ShapeDtypeStruct((NC, n, d), jnp.float32),
        scratch_types=[
            pltpu.VMEM((1, CH), jnp.int32),
            pltpu.VMEM((1, CH), jnp.int32),
            pltpu.VMEM((1, CH, d), jnp.float32),
            pltpu.VMEM_SHARED((n, d), jnp.float32),
            pltpu.SemaphoreType.DMA,
        ],
    )(h, edge_index, jnp.zeros((rpt, d), jnp.float32))

    out = pl.pallas_call(
        _fin_body,
        grid=(n // bm,),
        in_specs=[
            pl.BlockSpec((NC, bm, d), lambda i: (0, i, 0)),
            pl.BlockSpec((bm, 1), lambda i: (i, 0)),
            pl.BlockSpec((1, d), lambda i: (0, 0)),
            pl.BlockSpec((bm, d), lambda i: (i, 0)),
        ],
        out_specs=pl.BlockSpec((bm, d), lambda i: (i, 0)),
        out_shape=jax.ShapeDtypeStruct((n, d), jnp.float32),
    )(parts, deg_dst, b.reshape(1, d), feats)

    return out


# speedup vs baseline: 6.3428x; 6.3428x over previous
import functools

import jax
import jax.numpy as jnp
from jax import lax
from jax.experimental import pallas as pl
from jax.experimental.pallas import tpu as pltpu
from jax.experimental.pallas import tpu_sc as plsc

NC = 2
NS = 16
NW = NC * NS
CH = 128


def _hist_body(e, ncb, ncr, seg, edges, ones_h, zeros_h, out, sidx, didx,
               ones_v, acc_s, acc_d):
    cid = lax.axis_index("c")
    sid = lax.axis_index("s")
    wid = sid * NC + cid
    npad = NS * seg

    pltpu.sync_copy(ones_h, ones_v)
    pltpu.sync_copy(zeros_h, acc_s.at[pl.ds(sid * seg, seg)])
    pltpu.sync_copy(zeros_h, acc_d.at[pl.ds(sid * seg, seg)])
    plsc.subcore_barrier()

    nchunks = jnp.where(wid < ncr, ncb + 1, ncb)

    def body(i, carry):
        base = (wid + NW * i) * CH
        pltpu.sync_copy(edges.at[pl.ds(base, CH)], sidx.at[0])
        pltpu.sync_copy(edges.at[pl.ds(e + base, CH)], didx.at[0])
        pltpu.sync_copy(ones_v, acc_s.at[sidx.at[0]], add=True)
        pltpu.sync_copy(ones_v, acc_d.at[didx.at[0]], add=True)
        return carry

    lax.fori_loop(0, nchunks, body, 0)
    plsc.subcore_barrier()
    pltpu.sync_copy(acc_s.at[pl.ds(sid * seg, seg)],
                    out.at[pl.ds(cid * 2 * npad + sid * seg, seg)])
    pltpu.sync_copy(acc_d.at[pl.ds(sid * seg, seg)],
                    out.at[pl.ds((cid * 2 + 1) * npad + sid * seg, seg)])


def _agg_body(e, ncb, ncr, rpt, h, edges, zeros_a, out, sidx, didx, rows, acc,
              sem):
    cid = lax.axis_index("c")
    sid = lax.axis_index("s")
    wid = sid * NC + cid

    pltpu.sync_copy(zeros_a, acc.at[pl.ds(sid * rpt, rpt)])
    plsc.subcore_barrier()

    nchunks = jnp.where(wid < ncr, ncb + 1, ncb)

    def body(i, carry):
        base = (wid + NW * i) * CH
        pltpu.sync_copy(edges.at[pl.ds(base, CH)], sidx.at[0])
        pltpu.sync_copy(edges.at[pl.ds(e + base, CH)], didx.at[0])
        pltpu.async_copy(h.at[sidx.at[0]], rows.at[0], sem).wait()
        pltpu.sync_copy(rows.at[0], acc.at[didx.at[0]], add=True)
        return carry

    lax.fori_loop(0, nchunks, body, 0)
    plsc.subcore_barrier()
    pltpu.sync_copy(acc.at[pl.ds(sid * rpt, rpt)],
                    out.at[cid, pl.ds(sid * rpt, rpt)])


def _mm_body(deg_ref, x_ref, w_ref, o_ref):
    norm = lax.rsqrt(jnp.maximum(deg_ref[...], 1.0))
    o_ref[...] = jnp.dot(x_ref[...] * norm, w_ref[...],
                         preferred_element_type=jnp.float32)


def _fin_body(p_ref, deg_ref, b_ref, x_ref, o_ref):
    agg = p_ref[0] + p_ref[1]
    norm = lax.rsqrt(jnp.maximum(deg_ref[...], 1.0))
    o = jnp.maximum(agg * norm + b_ref[...], 0.0)
    o_ref[...] = o + x_ref[...]


def kernel(feats, edge_index, W, b):
    n, d = feats.shape
    e = edge_index.shape[1]
    assert e % CH == 0
    ec = e // CH
    ncb, ncr = ec // NW, ec % NW
    seg = ((n + NS * 128 - 1) // (NS * 128)) * 128
    npad = NS * seg
    rpt = seg
    npr = NS * rpt

    mesh = plsc.VectorSubcoreMesh(core_axis_name="c", subcore_axis_name="s")
    edge_flat = edge_index.reshape(2 * e)

    hist = pl.kernel(
        functools.partial(_hist_body, e, ncb, ncr, seg),
        mesh=mesh,
        out_type=jax.ShapeDtypeStruct((NC * 2 * npad,), jnp.float32),
        scratch_types=[
            pltpu.VMEM((1, CH), jnp.int32),
            pltpu.VMEM((1, CH), jnp.int32),
            pltpu.VMEM((CH,), jnp.float32),
            pltpu.VMEM_SHARED((npad,), jnp.float32),
            pltpu.VMEM_SHARED((npad,), jnp.float32),
        ],
    )(edge_flat, jnp.ones((CH,), jnp.float32), jnp.zeros((seg,), jnp.float32))
    hist = hist.reshape(NC, 2, npad)

    deg_src = (hist[0, 0, :n] + hist[1, 0, :n]).reshape(n, 1)
    deg_dst = (hist[0, 1, :n] + hist[1, 1, :n]).reshape(n, 1)

    bm = 2000
    assert n % bm == 0
    h = pl.pallas_call(
        _mm_body,
        grid=(n // bm,),
        in_specs=[
            pl.BlockSpec((bm, 1), lambda i: (i, 0)),
            pl.BlockSpec((bm, d), lambda i: (i, 0)),
            pl.BlockSpec((d, d), lambda i: (0, 0)),
        ],
        out_specs=pl.BlockSpec((bm, d), lambda i: (i, 0)),
        out_shape=jax.ShapeDtypeStruct((n, d), jnp.float32),
    )(deg_src, feats, W)

    parts = pl.kernel(
        functools.partial(_agg_body, e, ncb, ncr, rpt),
        mesh=mesh,
        out_type=jax.ShapeDtypeStruct((NC, npr, d), jnp.float32),
        scratch_types=[
            pltpu.VMEM((1, CH), jnp.int32),
            pltpu.VMEM((1, CH), jnp.int32),
            pltpu.VMEM((1, CH, d), jnp.float32),
            pltpu.VMEM_SHARED((npr, d), jnp.float32),
            pltpu.SemaphoreType.DMA,
        ],
    )(h, edge_flat, jnp.zeros((rpt, d), jnp.float32))

    out = pl.pallas_call(
        _fin_body,
        grid=(n // bm,),
        in_specs=[
            pl.BlockSpec((NC, bm, d), lambda i: (0, i, 0)),
            pl.BlockSpec((bm, 1), lambda i: (i, 0)),
            pl.BlockSpec((1, d), lambda i: (0, 0)),
            pl.BlockSpec((bm, d), lambda i: (i, 0)),
        ],
        out_specs=pl.BlockSpec((bm, d), lambda i: (i, 0)),
        out_shape=jax.ShapeDtypeStruct((n, d), jnp.float32),
    )(parts, deg_dst, b.reshape(1, d), feats)

    return out
